# baseline (speedup 1.0000x reference)
import functools

import jax
import jax.numpy as jnp
from jax import lax
from jax.experimental import pallas as pl
from jax.experimental.pallas import tpu as pltpu
from jax.experimental.pallas import tpu_sc as plsc

BN_EPS = 1e-5
NUM_G = 8
NC = 2
NS = 16
NW = NC * NS


CHUNK = 125
SB = 20
NSTRIP = 4


@functools.lru_cache(maxsize=None)
def _make_sc_aggregate(n, d):
    mesh = plsc.VectorSubcoreMesh(
        core_axis_name="c", subcore_axis_name="s", num_cores=NC,
        num_subcores=NS)
    rpt = ((n // NS) // 8) * 8
    rlast = n - rpt * (NS - 1)

    @functools.partial(
        pl.kernel,
        mesh=mesh,
        out_type=jax.ShapeDtypeStruct((NC, n, d), jnp.float32),
        scratch_types=[
            pltpu.VMEM_SHARED((n, d), jnp.float32),
            pltpu.VMEM((SB, CHUNK), jnp.int32),
            pltpu.VMEM((SB, CHUNK), jnp.int32),
            pltpu.VMEM((SB, CHUNK), jnp.int32),
            pltpu.VMEM((SB, CHUNK), jnp.int32),
            pltpu.VMEM((CHUNK, d), jnp.float32),
            pltpu.VMEM((CHUNK, d), jnp.float32),
            pltpu.SemaphoreType.DMA,
            pltpu.SemaphoreType.DMA,
            pltpu.SemaphoreType.DMA,
            pltpu.SemaphoreType.DMA,
            pltpu.SemaphoreType.DMA,
            pltpu.SemaphoreType.DMA,
        ],
    )
    def sc_agg(h_hbm, ei_hbm, zero_hbm, out_hbm,
               acc_s, sia_v, dia_v, sib_v, dib_v, rows0_v, rows1_v,
               sem_sa, sem_da, sem_sb, sem_db, semg0, semg1):
        c = lax.axis_index("c")
        s = lax.axis_index("s")
        wid = s * NC + c

        pltpu.async_copy(ei_hbm.at[0, wid, 0], sia_v, sem_sa)
        pltpu.async_copy(ei_hbm.at[1, wid, 0], dia_v, sem_da)
        pltpu.async_copy(ei_hbm.at[0, wid, 1], sib_v, sem_sb)
        pltpu.async_copy(ei_hbm.at[1, wid, 1], dib_v, sem_db)

        @pl.when(s < NS - 1)
        def _():
            pltpu.sync_copy(zero_hbm.at[pl.ds(s * rpt, rpt)],
                            acc_s.at[pl.ds(s * rpt, rpt)])

        @pl.when(s == NS - 1)
        def _():
            pltpu.sync_copy(zero_hbm.at[pl.ds((NS - 1) * rpt, rlast)],
                            acc_s.at[pl.ds((NS - 1) * rpt, rlast)])
        plsc.subcore_barrier()

        def strip_proc(si_v, di_v, nxt_si, tail_gate, tail_sem):
            def inner(u, carry):
                k = 2 * u
                pltpu.make_async_copy(
                    h_hbm.at[si_v.at[0]], rows0_v, semg0).wait()
                pltpu.sync_copy(rows0_v, acc_s.at[di_v.at[k]], add=True)

                @pl.when(k + 2 < SB)
                def _():
                    pltpu.async_copy(h_hbm.at[si_v.at[k + 2]], rows0_v, semg0)

                @pl.when(jnp.logical_and(k + 2 >= SB, tail_gate))
                def _():
                    pltpu.make_async_copy(
                        ei_hbm.at[0, wid, 0], nxt_si, tail_sem).wait()
                    pltpu.async_copy(h_hbm.at[nxt_si.at[0]], rows0_v, semg0)

                pltpu.make_async_copy(
                    h_hbm.at[si_v.at[1]], rows1_v, semg1).wait()
                pltpu.sync_copy(rows1_v, acc_s.at[di_v.at[k + 1]], add=True)

                @pl.when(k + 3 < SB)
                def _():
                    pltpu.async_copy(h_hbm.at[si_v.at[k + 3]], rows1_v, semg1)

                @pl.when(jnp.logical_and(k + 3 >= SB, tail_gate))
                def _():
                    pltpu.async_copy(h_hbm.at[nxt_si.at[1]], rows1_v, semg1)

                return carry

            lax.fori_loop(0, SB // 2, inner, 0)

        def outer(t, carry):
            st = 2 * t
            strip_proc(sia_v, dia_v, sib_v, st + 1 < NSTRIP, sem_sb)
            pltpu.make_async_copy(ei_hbm.at[1, wid, 0], dib_v, sem_db).wait()

            @pl.when(st + 2 < NSTRIP)
            def _():
                pltpu.async_copy(ei_hbm.at[0, wid, st + 2], sia_v, sem_sa)
                pltpu.async_copy(ei_hbm.at[1, wid, st + 2], dia_v, sem_da)

            strip_proc(sib_v, dib_v, sia_v, st + 2 < NSTRIP, sem_sa)

            @pl.when(st + 2 < NSTRIP)
            def _():
                pltpu.make_async_copy(
                    ei_hbm.at[1, wid, 0], dia_v, sem_da).wait()

            @pl.when(st + 3 < NSTRIP)
            def _():
                pltpu.async_copy(ei_hbm.at[0, wid, st + 3], sib_v, sem_sb)
                pltpu.async_copy(ei_hbm.at[1, wid, st + 3], dib_v, sem_db)

            return carry

        pltpu.make_async_copy(ei_hbm.at[0, wid, 0], sia_v, sem_sa).wait()
        pltpu.make_async_copy(ei_hbm.at[1, wid, 0], dia_v, sem_da).wait()
        pltpu.async_copy(h_hbm.at[sia_v.at[0]], rows0_v, semg0)
        pltpu.async_copy(h_hbm.at[sia_v.at[1]], rows1_v, semg1)
        lax.fori_loop(0, NSTRIP // 2, outer, 0)
        plsc.subcore_barrier()

        @pl.when(s < NS - 1)
        def _():
            pltpu.sync_copy(acc_s.at[pl.ds(s * rpt, rpt)],
                            out_hbm.at[c, pl.ds(s * rpt, rpt)])

        @pl.when(s == NS - 1)
        def _():
            pltpu.sync_copy(acc_s.at[pl.ds((NS - 1) * rpt, rlast)],
                            out_hbm.at[c, pl.ds((NS - 1) * rpt, rlast)])

    return sc_agg


def _bn_relu(h, g, be, n):
    mu = jnp.sum(h, axis=0, keepdims=True) * (1.0 / n)
    dlt = h - mu
    var = jnp.sum(dlt * dlt, axis=0, keepdims=True) * (1.0 / n)
    return jnp.maximum(g * dlt * lax.rsqrt(var + BN_EPS) + be, 0.0)


def _block0_body(x_ref, w_ref, b_ref, g_ref, be_ref, o_ref):
    n = x_ref.shape[0]
    h = jnp.dot(x_ref[...], w_ref[...], preferred_element_type=jnp.float32)
    o_ref[...] = _bn_relu(h + b_ref[...], g_ref[...], be_ref[...], n)


def _conv_body(h_ref, a0_ref, a1_ref, w_ref, b_ref, g_ref, be_ref, o_ref):
    n = h_ref.shape[0]
    t = h_ref[...] + a0_ref[...] + a1_ref[...]
    h = jnp.dot(t, w_ref[...], preferred_element_type=jnp.float32)
    o_ref[...] = _bn_relu(h + b_ref[...], g_ref[...], be_ref[...], n)


def _sigmoid(x):
    return 1.0 / (1.0 + jnp.exp(-x))


def _head_body(h_ref, a0_ref, a1_ref, wc_ref, bc_ref, gc_ref, bec_ref,
               w1_ref, b1_ref, g1_ref, be1_ref,
               w2_ref, b2_ref, g2_ref, be2_ref,
               w3_ref, b3_ref, batch_ref,
               pooled_ref, node_ref):
    n = h_ref.shape[0]
    t = h_ref[...] + a0_ref[...] + a1_ref[...]
    h = jnp.dot(t, wc_ref[...], preferred_element_type=jnp.float32)
    h = _bn_relu(h + bc_ref[...], gc_ref[...], bec_ref[...], n)
    m = jnp.dot(h, w1_ref[...], preferred_element_type=jnp.float32)
    m = _bn_relu(m + b1_ref[...], g1_ref[...], be1_ref[...], n)
    m = jnp.dot(m, w2_ref[...], preferred_element_type=jnp.float32)
    m = _bn_relu(m + b2_ref[...], g2_ref[...], be2_ref[...], n)
    node = jnp.dot(m, w3_ref[...], preferred_element_type=jnp.float32)
    node = node + b3_ref[...]
    gid = lax.broadcasted_iota(jnp.int32, (n, NUM_G), 1)
    mask = batch_ref[...] == gid
    vals = jnp.where(mask, node, -jnp.inf)
    pooled = jnp.max(vals, axis=0, keepdims=True)
    pooled_ref[...] = _sigmoid(pooled)
    node_ref[...] = _sigmoid(node)


def kernel(x, params, edge_index, batch):
    n, d = x.shape
    e = edge_index.shape[1]
    assert e % NW == 0
    epw = e // NW
    assert epw == NSTRIP * SB * CHUNK, "edge partitioning must be exact"

    ei = edge_index.reshape(2, NW, NSTRIP, SB, CHUNK)
    zero = jnp.zeros((n, d), jnp.float32)
    sc_agg = _make_sc_aggregate(n, d)

    def row(v):
        return v.reshape(1, -1)

    p = params['first']
    h = pl.pallas_call(
        _block0_body,
        out_shape=jax.ShapeDtypeStruct((n, d), jnp.float32),
    )(x, p['W'], row(p['b']), row(p['g']), row(p['be']))

    agg = sc_agg(h, ei, zero)
    p = params['conv0']
    h = pl.pallas_call(
        _conv_body,
        out_shape=jax.ShapeDtypeStruct((n, d), jnp.float32),
    )(h, agg[0], agg[1], p['W'], row(p['b']), row(p['g']), row(p['be']))

    agg = sc_agg(h, ei, zero)
    pc = params['conv1']
    pm = params['mlp']
    pooled, node = pl.pallas_call(
        _head_body,
        out_shape=[
            jax.ShapeDtypeStruct((1, NUM_G), jnp.float32),
            jax.ShapeDtypeStruct((n, 1), jnp.float32),
        ],
    )(h, agg[0], agg[1], pc['W'], row(pc['b']), row(pc['g']), row(pc['be']),
      pm['W1'], row(pm['b1']), row(pm['g1']), row(pm['be1']),
      pm['W2'], row(pm['b2']), row(pm['g2']), row(pm['be2']),
      pm['W3'], row(pm['b3']), batch.reshape(n, 1))

    return pooled.reshape(NUM_G, 1), node


# speedup vs baseline: 1.0084x; 1.0084x over previous
import functools

import jax
import jax.numpy as jnp
from jax import lax
from jax.experimental import pallas as pl
from jax.experimental.pallas import tpu as pltpu
from jax.experimental.pallas import tpu_sc as plsc

BN_EPS = 1e-5
NUM_G = 8
NC = 2
NS = 16
NW = NC * NS


CHUNK = 125
SB = 20
NSTRIP = 4


@functools.lru_cache(maxsize=None)
def _make_sc_aggregate(n, d):
    mesh = plsc.VectorSubcoreMesh(
        core_axis_name="c", subcore_axis_name="s", num_cores=NC,
        num_subcores=NS)
    rpt = ((n // NS) // 8) * 8
    rlast = n - rpt * (NS - 1)

    @functools.partial(
        pl.kernel,
        mesh=mesh,
        out_type=jax.ShapeDtypeStruct((NC, n, d), jnp.float32),
        scratch_types=[
            pltpu.VMEM_SHARED((n, d), jnp.float32),
            pltpu.VMEM((SB, CHUNK), jnp.int32),
            pltpu.VMEM((SB, CHUNK), jnp.int32),
            pltpu.VMEM((SB, CHUNK), jnp.int32),
            pltpu.VMEM((SB, CHUNK), jnp.int32),
            pltpu.VMEM((CHUNK, d), jnp.float32),
            pltpu.VMEM((CHUNK, d), jnp.float32),
            pltpu.SemaphoreType.DMA,
            pltpu.SemaphoreType.DMA,
            pltpu.SemaphoreType.DMA,
            pltpu.SemaphoreType.DMA,
            pltpu.SemaphoreType.DMA,
            pltpu.SemaphoreType.DMA,
        ],
    )
    def sc_agg(h_hbm, ei_hbm, zero_hbm, out_hbm,
               acc_s, sia_v, dia_v, sib_v, dib_v, rows0_v, rows1_v,
               sem_sa, sem_da, sem_sb, sem_db, semg0, semg1):
        c = lax.axis_index("c")
        s = lax.axis_index("s")
        wid = s * NC + c

        pltpu.async_copy(ei_hbm.at[0, wid, 0], sia_v, sem_sa)
        pltpu.async_copy(ei_hbm.at[1, wid, 0], dia_v, sem_da)
        pltpu.async_copy(ei_hbm.at[0, wid, 1], sib_v, sem_sb)
        pltpu.async_copy(ei_hbm.at[1, wid, 1], dib_v, sem_db)

        init_hbm = [h_hbm, zero_hbm]

        for cc in range(NC):
            @pl.when(jnp.logical_and(c == cc, s < NS - 1))
            def _(cc=cc):
                pltpu.sync_copy(init_hbm[cc].at[pl.ds(s * rpt, rpt)],
                                acc_s.at[pl.ds(s * rpt, rpt)])

            @pl.when(jnp.logical_and(c == cc, s == NS - 1))
            def _(cc=cc):
                pltpu.sync_copy(init_hbm[cc].at[pl.ds((NS - 1) * rpt, rlast)],
                                acc_s.at[pl.ds((NS - 1) * rpt, rlast)])
        plsc.subcore_barrier()

        def strip_proc(si_v, di_v, nxt_si, tail_gate, tail_sem):
            def inner(u, carry):
                k = 2 * u
                pltpu.make_async_copy(
                    h_hbm.at[si_v.at[0]], rows0_v, semg0).wait()
                pltpu.sync_copy(rows0_v, acc_s.at[di_v.at[k]], add=True)

                @pl.when(k + 2 < SB)
                def _():
                    pltpu.async_copy(h_hbm.at[si_v.at[k + 2]], rows0_v, semg0)

                @pl.when(jnp.logical_and(k + 2 >= SB, tail_gate))
                def _():
                    pltpu.make_async_copy(
                        ei_hbm.at[0, wid, 0], nxt_si, tail_sem).wait()
                    pltpu.async_copy(h_hbm.at[nxt_si.at[0]], rows0_v, semg0)

                pltpu.make_async_copy(
                    h_hbm.at[si_v.at[1]], rows1_v, semg1).wait()
                pltpu.sync_copy(rows1_v, acc_s.at[di_v.at[k + 1]], add=True)

                @pl.when(k + 3 < SB)
                def _():
                    pltpu.async_copy(h_hbm.at[si_v.at[k + 3]], rows1_v, semg1)

                @pl.when(jnp.logical_and(k + 3 >= SB, tail_gate))
                def _():
                    pltpu.async_copy(h_hbm.at[nxt_si.at[1]], rows1_v, semg1)

                return carry

            lax.fori_loop(0, SB // 2, inner, 0)

        def outer(t, carry):
            st = 2 * t
            strip_proc(sia_v, dia_v, sib_v, st + 1 < NSTRIP, sem_sb)
            pltpu.make_async_copy(ei_hbm.at[1, wid, 0], dib_v, sem_db).wait()

            @pl.when(st + 2 < NSTRIP)
            def _():
                pltpu.async_copy(ei_hbm.at[0, wid, st + 2], sia_v, sem_sa)
                pltpu.async_copy(ei_hbm.at[1, wid, st + 2], dia_v, sem_da)

            strip_proc(sib_v, dib_v, sia_v, st + 2 < NSTRIP, sem_sa)

            @pl.when(st + 2 < NSTRIP)
            def _():
                pltpu.make_async_copy(
                    ei_hbm.at[1, wid, 0], dia_v, sem_da).wait()

            @pl.when(st + 3 < NSTRIP)
            def _():
                pltpu.async_copy(ei_hbm.at[0, wid, st + 3], sib_v, sem_sb)
                pltpu.async_copy(ei_hbm.at[1, wid, st + 3], dib_v, sem_db)

            return carry

        pltpu.make_async_copy(ei_hbm.at[0, wid, 0], sia_v, sem_sa).wait()
        pltpu.make_async_copy(ei_hbm.at[1, wid, 0], dia_v, sem_da).wait()
        pltpu.async_copy(h_hbm.at[sia_v.at[0]], rows0_v, semg0)
        pltpu.async_copy(h_hbm.at[sia_v.at[1]], rows1_v, semg1)
        lax.fori_loop(0, NSTRIP // 2, outer, 0)
        plsc.subcore_barrier()

        @pl.when(s < NS - 1)
        def _():
            pltpu.sync_copy(acc_s.at[pl.ds(s * rpt, rpt)],
                            out_hbm.at[c, pl.ds(s * rpt, rpt)])

        @pl.when(s == NS - 1)
        def _():
            pltpu.sync_copy(acc_s.at[pl.ds((NS - 1) * rpt, rlast)],
                            out_hbm.at[c, pl.ds((NS - 1) * rpt, rlast)])

    return sc_agg


def _bn_relu(h, g, be, n):
    mu = jnp.sum(h, axis=0, keepdims=True) * (1.0 / n)
    dlt = h - mu
    var = jnp.sum(dlt * dlt, axis=0, keepdims=True) * (1.0 / n)
    return jnp.maximum(g * dlt * lax.rsqrt(var + BN_EPS) + be, 0.0)


def _block0_body(x_ref, w_ref, b_ref, g_ref, be_ref, o_ref):
    n = x_ref.shape[0]
    h = jnp.dot(x_ref[...], w_ref[...], preferred_element_type=jnp.float32)
    o_ref[...] = _bn_relu(h + b_ref[...], g_ref[...], be_ref[...], n)


def _conv_body(a0_ref, a1_ref, w_ref, b_ref, g_ref, be_ref, o_ref):
    n = a0_ref.shape[0]
    t = a0_ref[...] + a1_ref[...]
    h = jnp.dot(t, w_ref[...], preferred_element_type=jnp.float32)
    o_ref[...] = _bn_relu(h + b_ref[...], g_ref[...], be_ref[...], n)


def _sigmoid(x):
    return 1.0 / (1.0 + jnp.exp(-x))


def _head_body(a0_ref, a1_ref, wc_ref, bc_ref, gc_ref, bec_ref,
               w1_ref, b1_ref, g1_ref, be1_ref,
               w2_ref, b2_ref, g2_ref, be2_ref,
               w3_ref, b3_ref, batch_ref,
               pooled_ref, node_ref):
    n = a0_ref.shape[0]
    t = a0_ref[...] + a1_ref[...]
    h = jnp.dot(t, wc_ref[...], preferred_element_type=jnp.float32)
    h = _bn_relu(h + bc_ref[...], gc_ref[...], bec_ref[...], n)
    m = jnp.dot(h, w1_ref[...], preferred_element_type=jnp.float32)
    m = _bn_relu(m + b1_ref[...], g1_ref[...], be1_ref[...], n)
    m = jnp.dot(m, w2_ref[...], preferred_element_type=jnp.float32)
    m = _bn_relu(m + b2_ref[...], g2_ref[...], be2_ref[...], n)
    node = jnp.dot(m, w3_ref[...], preferred_element_type=jnp.float32)
    node = node + b3_ref[...]
    gid = lax.broadcasted_iota(jnp.int32, (n, NUM_G), 1)
    mask = batch_ref[...] == gid
    vals = jnp.where(mask, node, -jnp.inf)
    pooled = jnp.max(vals, axis=0, keepdims=True)
    pooled_ref[...] = _sigmoid(pooled)
    node_ref[...] = _sigmoid(node)


def kernel(x, params, edge_index, batch):
    n, d = x.shape
    e = edge_index.shape[1]
    assert e % NW == 0
    epw = e // NW
    assert epw == NSTRIP * SB * CHUNK, "edge partitioning must be exact"

    ei = edge_index.reshape(2, NW, NSTRIP, SB, CHUNK)
    zero = jnp.zeros((n, d), jnp.float32)
    sc_agg = _make_sc_aggregate(n, d)

    def row(v):
        return v.reshape(1, -1)

    p = params['first']
    h = pl.pallas_call(
        _block0_body,
        out_shape=jax.ShapeDtypeStruct((n, d), jnp.float32),
    )(x, p['W'], row(p['b']), row(p['g']), row(p['be']))

    agg = sc_agg(h, ei, zero)
    p = params['conv0']
    h = pl.pallas_call(
        _conv_body,
        out_shape=jax.ShapeDtypeStruct((n, d), jnp.float32),
    )(agg[0], agg[1], p['W'], row(p['b']), row(p['g']), row(p['be']))

    agg = sc_agg(h, ei, zero)
    pc = params['conv1']
    pm = params['mlp']
    pooled, node = pl.pallas_call(
        _head_body,
        out_shape=[
            jax.ShapeDtypeStruct((1, NUM_G), jnp.float32),
            jax.ShapeDtypeStruct((n, 1), jnp.float32),
        ],
    )(agg[0], agg[1], pc['W'], row(pc['b']), row(pc['g']), row(pc['be']),
      pm['W1'], row(pm['b1']), row(pm['g1']), row(pm['be1']),
      pm['W2'], row(pm['b2']), row(pm['g2']), row(pm['be2']),
      pm['W3'], row(pm['b3']), batch.reshape(n, 1))

    return pooled.reshape(NUM_G, 1), node
